# baseline (speedup 1.0000x reference)
import functools

import jax
import jax.numpy as jnp
from jax import lax
from jax.experimental import pallas as pl
from jax.experimental.pallas import tpu as pltpu
from jax.experimental.pallas import tpu_sc as plsc

_NC = 2
_NS = 16
_NW = _NC * _NS
_L = 16

_ROWS = 4096
_W = 4096
_F = 1024
_RPW = _ROWS // _NW
_CB = 16


def _sc_body(x_hbm, m_hbm, out_hbm, mout_hbm, in_v, out_v, m_v, mo_v):
    wid = lax.axis_index("s") * _NC + lax.axis_index("c")
    base = wid * _RPW

    for k in range(4):
        pltpu.sync_copy(
            m_hbm.at[pl.ds(k * _ROWS + base, _RPW)], m_v.at[k]
        )

    def mask_blk(j, _):
        acc = m_v[0, pl.ds(j * _L, _L)]
        for k in range(1, 4):
            acc = jnp.minimum(acc, m_v[k, pl.ds(j * _L, _L)])
        mo_v[pl.ds(j * _L, _L)] = acc
        return 0

    lax.fori_loop(0, _RPW // _L, mask_blk, 0, unroll=True)
    pltpu.sync_copy(mo_v, mout_hbm.at[pl.ds(base, _RPW)])

    def chunk(ci, _):
        r0 = base + ci * _CB
        pltpu.sync_copy(x_hbm.at[pl.ds(r0, _CB)], in_v)

        def row(r, _):
            def col(j, _):
                c = j * _L
                a = in_v[r, pl.ds(c, _L)] + in_v[r, pl.ds(_F + c, _L)]
                a = a + in_v[r, pl.ds(2 * _F + c, _L)]
                a = a + in_v[r, pl.ds(3 * _F + c, _L)]
                out_v[r, pl.ds(c, _L)] = a * 0.25
                return 0

            lax.fori_loop(0, _F // _L, col, 0, unroll=True)
            return 0

        lax.fori_loop(0, _CB, row, 0)
        pltpu.sync_copy(out_v, out_hbm.at[pl.ds(r0, _CB)])
        return 0

    lax.fori_loop(0, _RPW // _CB, chunk, 0)


_sc_call = functools.partial(
    pl.kernel,
    out_type=[
        jax.ShapeDtypeStruct((_ROWS, _F), jnp.float32),
        jax.ShapeDtypeStruct((_ROWS,), jnp.int32),
    ],
    mesh=plsc.VectorSubcoreMesh(core_axis_name="c", subcore_axis_name="s"),
    scratch_types=[
        pltpu.VMEM((_CB, _W), jnp.float32),
        pltpu.VMEM((_CB, _F), jnp.float32),
        pltpu.VMEM((4, _RPW), jnp.int32),
        pltpu.VMEM((_RPW,), jnp.int32),
    ],
)(_sc_body)


def kernel(dense_x, dense_padding_mask):
    bsz, tsz, fsz = dense_x.shape
    gs = 4
    tn = tsz // gs

    x2 = dense_x.reshape(bsz * tn, gs * fsz)
    m4 = (
        dense_padding_mask.reshape(bsz * tn, gs)
        .astype(jnp.int32)
        .T.reshape(gs * bsz * tn)
    )

    out, mout = _sc_call(x2, m4)
    return (
        out.reshape(bsz, tn, fsz),
        mout.reshape(bsz, tn).astype(jnp.bool_),
    )


# speedup vs baseline: 1.5189x; 1.5189x over previous
import functools

import jax
import jax.numpy as jnp
from jax import lax
from jax.experimental import pallas as pl
from jax.experimental.pallas import tpu as pltpu
from jax.experimental.pallas import tpu_sc as plsc

_NC = 2
_NS = 16
_NW = _NC * _NS
_L = 16

_ROWS = 4096
_W = 4096
_F = 1024
_RPW = _ROWS // _NW
_CB = 8
_NCHUNK = _RPW // _CB


def _sc_body(
    x_hbm,
    m_hbm,
    out_hbm,
    mout_hbm,
    in_v0,
    in_v1,
    out_v0,
    out_v1,
    m_v,
    mo_v,
    si0,
    si1,
    so0,
    so1,
):
    wid = lax.axis_index("s") * _NC + lax.axis_index("c")
    base = wid * _RPW

    in_bufs = (in_v0, in_v1)
    out_bufs = (out_v0, out_v1)
    sin = (si0, si1)
    sout = (so0, so1)

    for k in range(4):
        pltpu.sync_copy(m_hbm.at[pl.ds(k * _ROWS + base, _RPW)], m_v.at[k])

    def mask_blk(j, _):
        acc = m_v[0, pl.ds(j * _L, _L)]
        for k in range(1, 4):
            acc = jnp.minimum(acc, m_v[k, pl.ds(j * _L, _L)])
        mo_v[pl.ds(j * _L, _L)] = acc
        return 0

    lax.fori_loop(0, _RPW // _L, mask_blk, 0, unroll=True)
    pltpu.sync_copy(mo_v, mout_hbm.at[pl.ds(base, _RPW)])

    in_copies = [None] * _NCHUNK
    out_copies = [None] * _NCHUNK
    in_copies[0] = pltpu.async_copy(
        x_hbm.at[pl.ds(base, _CB)], in_bufs[0], sin[0]
    )

    def compute(b):
        in_b, out_b = in_bufs[b], out_bufs[b]

        def row(r, _):
            def col(j, _):
                c = j * _L
                a = in_b[r, pl.ds(c, _L)] + in_b[r, pl.ds(_F + c, _L)]
                a = a + in_b[r, pl.ds(2 * _F + c, _L)]
                a = a + in_b[r, pl.ds(3 * _F + c, _L)]
                out_b[r, pl.ds(c, _L)] = a * 0.25
                return 0

            lax.fori_loop(0, _F // _L, col, 0, unroll=8)
            return 0

        lax.fori_loop(0, _CB, row, 0)

    for ci in range(_NCHUNK):
        b = ci % 2
        nb = (ci + 1) % 2
        if ci + 1 < _NCHUNK:
            in_copies[ci + 1] = pltpu.async_copy(
                x_hbm.at[pl.ds(base + (ci + 1) * _CB, _CB)], in_bufs[nb], sin[nb]
            )
        in_copies[ci].wait()
        if ci >= 2:
            out_copies[ci - 2].wait()
        compute(b)
        out_copies[ci] = pltpu.async_copy(
            out_bufs[b], out_hbm.at[pl.ds(base + ci * _CB, _CB)], sout[b]
        )
    out_copies[_NCHUNK - 2].wait()
    out_copies[_NCHUNK - 1].wait()


_sc_call = functools.partial(
    pl.kernel,
    out_type=[
        jax.ShapeDtypeStruct((_ROWS, _F), jnp.float32),
        jax.ShapeDtypeStruct((_ROWS,), jnp.int32),
    ],
    mesh=plsc.VectorSubcoreMesh(core_axis_name="c", subcore_axis_name="s"),
    scratch_types=[
        pltpu.VMEM((_CB, _W), jnp.float32),
        pltpu.VMEM((_CB, _W), jnp.float32),
        pltpu.VMEM((_CB, _F), jnp.float32),
        pltpu.VMEM((_CB, _F), jnp.float32),
        pltpu.VMEM((4, _RPW), jnp.int32),
        pltpu.VMEM((_RPW,), jnp.int32),
        pltpu.SemaphoreType.DMA,
        pltpu.SemaphoreType.DMA,
        pltpu.SemaphoreType.DMA,
        pltpu.SemaphoreType.DMA,
    ],
)(_sc_body)


def kernel(dense_x, dense_padding_mask):
    bsz, tsz, fsz = dense_x.shape
    gs = 4
    tn = tsz // gs

    x2 = dense_x.reshape(bsz * tn, gs * fsz)
    m4 = (
        dense_padding_mask.reshape(bsz * tn, gs)
        .astype(jnp.int32)
        .T.reshape(gs * bsz * tn)
    )

    out, mout = _sc_call(x2, m4)
    return (
        out.reshape(bsz, tn, fsz),
        mout.reshape(bsz, tn).astype(jnp.bool_),
    )
